# baseline (speedup 1.0000x reference)
import jax
import jax.numpy as jnp
from jax import lax
from jax.experimental import pallas as pl
from jax.experimental.pallas import tpu as pltpu
from jax.experimental.pallas import tpu_sc as plsc

B, L, D, N = 4, 2048, 4096, 16
NEW_L = L + N
NC, NS = 2, 16
NW = NC * NS
SUBS_PER_B = NW // B
LANES = 16

H = 2
DW = D // H
HL = L * H
HNEW = NEW_L * H
HR_PER_W = HL // SUBS_PER_B
NCH = HR_PER_W // LANES
NPAIR = NCH // 2


def _sc_body(emb_hbm, mem_hbm, pos_hbm, out_hbm,
             buf0, buf1, pos_v, g0, g1, s0, s1):
    c = lax.axis_index("c")
    s = lax.axis_index("s")
    wid = c * NS + s
    b = wid // SUBS_PER_B
    sub = wid % SUBS_PER_B

    pltpu.sync_copy(pos_hbm, pos_v)
    pos_all = pos_v[...]
    pos_vec = pos_all.at[jnp.full((LANES,), b, jnp.int32)].get(
        mode="promise_in_bounds")
    pos_h = pos_vec * H

    base_h = sub * HR_PER_W
    src0 = b * HL + base_h
    outb = b * HNEW
    iota = lax.iota(jnp.int32, LANES)

    def dst(ch):
        hh = base_h + ch * LANES + iota
        return outb + jnp.where(hh < pos_h, hh, hh + N * H)

    def start_gather(ch, buf, sem):
        r = pl.multiple_of(src0 + ch * LANES, LANES)
        pltpu.async_copy(emb_hbm.at[pl.ds(r, LANES)], buf, sem)

    def start_scatter(ch, buf, sem):
        pltpu.async_copy(buf, out_hbm.at[dst(ch)], sem)

    def wait_gather(buf, sem):
        pltpu.make_async_copy(emb_hbm.at[pl.ds(0, LANES)], buf, sem).wait()

    def wait_scatter(buf, sem):
        pltpu.make_async_copy(buf, out_hbm.at[pl.ds(0, LANES)], sem).wait()

    start_gather(0, buf0, g0)

    def body(g, _):
        c0 = 2 * g
        c1 = c0 + 1

        @pl.when(g > 0)
        def _():
            wait_scatter(buf1, s1)
        start_gather(c1, buf1, g1)
        wait_gather(buf0, g0)
        start_scatter(c0, buf0, s0)

        wait_scatter(buf0, s0)

        @pl.when(g < NPAIR - 1)
        def _():
            start_gather(c1 + 1, buf0, g0)
        wait_gather(buf1, g1)
        start_scatter(c1, buf1, s1)
        return 0
    lax.fori_loop(0, NPAIR, body, 0)
    wait_scatter(buf1, s1)

    @pl.when(sub == 0)
    def _():
        mb = b * N * H
        for k in range(2):
            pltpu.sync_copy(mem_hbm.at[pl.ds(mb + k * LANES, LANES)], buf0)
            dstm = outb + pos_h + k * LANES + iota
            pltpu.sync_copy(buf0, out_hbm.at[dstm])


def _mask_body(am_ref, pos_ref, out_ref):
    j = lax.broadcasted_iota(jnp.int32, (B, NEW_L), 1)
    pos = pos_ref[...].reshape(B, 1)
    am = am_ref[...]
    zpad = jnp.zeros((B, N), jnp.float32)
    am_lo = jnp.concatenate([am, zpad], axis=1)
    am_hi = jnp.concatenate([zpad, am], axis=1)
    out_ref[...] = jnp.where(
        j < pos, am_lo, jnp.where(j >= pos + N, am_hi,
                                  jnp.ones((B, NEW_L), jnp.float32)))


@jax.jit
def kernel(inputs_embeds, memory, attention_mask, injection_positions):
    emb_flat = inputs_embeds.reshape(B * HL, DW)
    mem_flat = memory.reshape(B * N * H, DW)
    am = attention_mask.astype(jnp.float32)
    pos32 = injection_positions.astype(jnp.int32)
    pos_pad = jnp.zeros((LANES,), jnp.int32).at[:B].set(pos32)

    mesh = plsc.VectorSubcoreMesh(core_axis_name="c", subcore_axis_name="s",
                                  num_cores=NC, num_subcores=NS)
    run = pl.kernel(
        _sc_body,
        out_type=jax.ShapeDtypeStruct((B * HNEW, DW), jnp.float32),
        mesh=mesh,
        scratch_types=[
            pltpu.VMEM((LANES, DW), jnp.float32),
            pltpu.VMEM((LANES, DW), jnp.float32),
            pltpu.VMEM((LANES,), jnp.int32),
            pltpu.SemaphoreType.DMA,
            pltpu.SemaphoreType.DMA,
            pltpu.SemaphoreType.DMA,
            pltpu.SemaphoreType.DMA,
        ],
    )
    out_flat = run(emb_flat, mem_flat, pos_pad)

    new_mask = pl.pallas_call(
        _mask_body,
        out_shape=jax.ShapeDtypeStruct((B, NEW_L), jnp.float32),
    )(am, pos32)

    return out_flat.reshape(B, NEW_L, D), new_mask


# speedup vs baseline: 3.1138x; 3.1138x over previous
import jax
import jax.numpy as jnp
from jax import lax
from jax.experimental import pallas as pl
from jax.experimental.pallas import tpu as pltpu
from jax.experimental.pallas import tpu_sc as plsc

B, L, D, N = 4, 2048, 4096, 16
NEW_L = L + N
NC, NS = 2, 16
NW = NC * NS
SUBS_PER_B = NW // B
ROWS_PER_W = L // SUBS_PER_B
CHUNK = 16
NCHUNK = ROWS_PER_W // CHUNK
LANES = 16


def _sc_body(emb_hbm, mem_hbm, pos_hbm, out_hbm, buf, pos_v):
    c = lax.axis_index("c")
    s = lax.axis_index("s")
    wid = c * NS + s
    b = wid // SUBS_PER_B
    sub = wid % SUBS_PER_B

    pltpu.sync_copy(pos_hbm, pos_v)
    pos_slice = pos_v[pl.ds(b, LANES)]
    pos_s = pos_slice[0]
    pos_vec = jnp.full((LANES,), pos_s, jnp.int32)

    base_local = sub * ROWS_PER_W
    src_base = b * L + base_local
    out_base = b * NEW_L
    iota = lax.iota(jnp.int32, LANES)

    def copy_body(i, _):
        row0 = base_local + i * CHUNK
        r = pl.multiple_of(src_base + i * CHUNK, CHUNK)
        pltpu.sync_copy(emb_hbm.at[pl.ds(r, CHUNK)], buf)

        straddles = jnp.logical_and(row0 < pos_s, pos_s < row0 + CHUNK)

        @pl.when(jnp.logical_not(straddles))
        def _():
            shift = jnp.where(row0 >= pos_s, N, 0)
            dst0 = pl.multiple_of(out_base + row0 + shift, 8)
            pltpu.sync_copy(buf, out_hbm.at[pl.ds(dst0, CHUNK)])

        @pl.when(straddles)
        def _():
            j = row0 + iota
            dst = out_base + jnp.where(j < pos_vec, j, j + N)
            pltpu.sync_copy(buf, out_hbm.at[dst])
        return 0
    lax.fori_loop(0, NCHUNK, copy_body, 0)

    @pl.when(sub == 0)
    def _():
        mrow0 = pl.multiple_of(b * N, N)
        pltpu.sync_copy(mem_hbm.at[pl.ds(mrow0, N)], buf)
        dstm = out_base + pos_vec + iota
        pltpu.sync_copy(buf, out_hbm.at[dstm])


def _mask_body(am_ref, pos_ref, out_ref):
    j = lax.broadcasted_iota(jnp.int32, (B, NEW_L), 1)
    pos = pos_ref[...].reshape(B, 1)
    am = am_ref[...]
    zpad = jnp.zeros((B, N), jnp.float32)
    am_lo = jnp.concatenate([am, zpad], axis=1)
    am_hi = jnp.concatenate([zpad, am], axis=1)
    out_ref[...] = jnp.where(
        j < pos, am_lo, jnp.where(j >= pos + N, am_hi,
                                  jnp.ones((B, NEW_L), jnp.float32)))


@jax.jit
def kernel(inputs_embeds, memory, attention_mask, injection_positions):
    emb_flat = inputs_embeds.reshape(B * L, D)
    mem_flat = memory.reshape(B * N, D)
    am = attention_mask.astype(jnp.float32)
    pos32 = injection_positions.astype(jnp.int32)
    pos_pad = jnp.zeros((2 * LANES,), jnp.int32).at[:B].set(pos32)

    mesh = plsc.VectorSubcoreMesh(core_axis_name="c", subcore_axis_name="s",
                                  num_cores=NC, num_subcores=NS)
    run = pl.kernel(
        _sc_body,
        out_type=jax.ShapeDtypeStruct((B * NEW_L, D), jnp.float32),
        mesh=mesh,
        scratch_types=[
            pltpu.VMEM((CHUNK, D), jnp.float32),
            pltpu.VMEM((2 * LANES,), jnp.int32),
        ],
    )
    out_flat = run(emb_flat, mem_flat, pos_pad)

    new_mask = pl.pallas_call(
        _mask_body,
        out_shape=jax.ShapeDtypeStruct((B, NEW_L), jnp.float32),
    )(am, pos32)

    return out_flat.reshape(B, NEW_L, D), new_mask


# speedup vs baseline: 3.2864x; 1.0554x over previous
import jax
import jax.numpy as jnp
from jax import lax
from jax.experimental import pallas as pl
from jax.experimental.pallas import tpu as pltpu
from jax.experimental.pallas import tpu_sc as plsc

B, L, D, N = 4, 2048, 4096, 16
NEW_L = L + N
NC, NS = 2, 16
NW = NC * NS
SUBS_PER_B = NW // B
ROWS_PER_W = L // SUBS_PER_B
CHUNK = 8
NCHUNK = ROWS_PER_W // CHUNK
NSLOT = 3
LANES = 16


def _sc_body(emb_hbm, mem_hbm, pos_hbm, out_hbm, buf, pos_v, gsems, ssems):
    c = lax.axis_index("c")
    s = lax.axis_index("s")
    wid = c * NS + s
    b = wid // SUBS_PER_B
    sub = wid % SUBS_PER_B

    pltpu.sync_copy(pos_hbm, pos_v)
    pos_s = pos_v[pl.ds(b, LANES)][0]
    pos_vec = jnp.full((LANES,), pos_s, jnp.int32)

    base_local = sub * ROWS_PER_W
    src_base = b * L + base_local
    out_base = b * NEW_L
    iota = lax.iota(jnp.int32, LANES)

    slots = [buf.at[pl.ds(k * CHUNK, CHUNK)] for k in range(NSLOT)]

    def start_gather(i):
        r = pl.multiple_of(src_base + i * CHUNK, CHUNK)
        pltpu.async_copy(emb_hbm.at[pl.ds(r, CHUNK)], slots[i % NSLOT],
                         gsems.at[i % NSLOT])

    def wait_gather(i):
        pltpu.make_async_copy(emb_hbm.at[pl.ds(0, CHUNK)], slots[i % NSLOT],
                              gsems.at[i % NSLOT]).wait()

    def scatter_starts(i, fn):
        row0 = base_local + i * CHUNK
        sl = slots[i % NSLOT]
        sem = ssems.at[i % NSLOT]

        @pl.when(row0 < pos_s)
        def _():
            fn(sl, pl.multiple_of(out_base + row0, 8), sem)

        @pl.when(row0 + CHUNK > pos_s)
        def _():
            fn(sl, pl.multiple_of(out_base + row0 + N, 8), sem)

    def start_scatter(i):
        scatter_starts(
            i, lambda sl, dst0, sem:
            pltpu.async_copy(sl, out_hbm.at[pl.ds(dst0, CHUNK)], sem))

    def wait_scatter(i):
        scatter_starts(
            i, lambda sl, dst0, sem:
            pltpu.make_async_copy(sl, out_hbm.at[pl.ds(dst0, CHUNK)],
                                  sem).wait())

    start_gather(0)
    start_gather(1)
    for i in range(NCHUNK):
        wait_gather(i)
        start_scatter(i)
        if i >= 1:
            wait_scatter(i - 1)
        if i + 2 < NCHUNK:
            start_gather(i + 2)
    wait_scatter(NCHUNK - 1)

    @pl.when(sub == pos_s // ROWS_PER_W)
    def _():
        mrow0 = pl.multiple_of(b * N, N)
        stage = buf.at[pl.ds(0, N)]
        pltpu.sync_copy(mem_hbm.at[pl.ds(mrow0, N)], stage)
        dstm = out_base + pos_vec + iota
        pltpu.sync_copy(stage, out_hbm.at[dstm])


def _mask_body(am_ref, pos_ref, out_ref):
    j = lax.broadcasted_iota(jnp.int32, (B, NEW_L), 1)
    pos = pos_ref[...].reshape(B, 1)
    am = am_ref[...]
    zpad = jnp.zeros((B, N), jnp.float32)
    am_lo = jnp.concatenate([am, zpad], axis=1)
    am_hi = jnp.concatenate([zpad, am], axis=1)
    out_ref[...] = jnp.where(
        j < pos, am_lo, jnp.where(j >= pos + N, am_hi,
                                  jnp.ones((B, NEW_L), jnp.float32)))


@jax.jit
def kernel(inputs_embeds, memory, attention_mask, injection_positions):
    emb_flat = inputs_embeds.reshape(B * L, D)
    mem_flat = memory.reshape(B * N, D)
    am = attention_mask.astype(jnp.float32)
    pos32 = injection_positions.astype(jnp.int32)
    pos_pad = jnp.zeros((2 * LANES,), jnp.int32).at[:B].set(pos32)

    mesh = plsc.VectorSubcoreMesh(core_axis_name="c", subcore_axis_name="s",
                                  num_cores=NC, num_subcores=NS)
    run = pl.kernel(
        _sc_body,
        out_type=jax.ShapeDtypeStruct((B * NEW_L, D), jnp.float32),
        mesh=mesh,
        scratch_types=[
            pltpu.VMEM((NSLOT * CHUNK, D), jnp.float32),
            pltpu.VMEM((2 * LANES,), jnp.int32),
            pltpu.SemaphoreType.DMA((NSLOT,)),
            pltpu.SemaphoreType.DMA((NSLOT,)),
        ],
    )
    out_flat = run(emb_flat, mem_flat, pos_pad)

    new_mask = pl.pallas_call(
        _mask_body,
        out_shape=jax.ShapeDtypeStruct((B, NEW_L), jnp.float32),
    )(am, pos32)

    return out_flat.reshape(B, NEW_L, D), new_mask
